# Initial kernel scaffold; baseline (speedup 1.0000x reference)
import functools

import jax
import jax.numpy as jnp
from jax.experimental import pallas as pl

_TOP_K = 8


def _gate_body(x_ref, wt_ref, sm_ref, idx_ref):
    logits = jnp.dot(x_ref[...], wt_ref[...], preferred_element_type=jnp.float32)
    bt, ne = logits.shape
    col = jax.lax.broadcasted_iota(jnp.int32, (bt, ne), 1)
    vals = logits
    scores = []
    indices = []
    neg_inf = jnp.float32(-jnp.inf)
    for _ in range(_TOP_K):
        m = jnp.max(vals, axis=1, keepdims=True)
        is_max = vals == m
        ind = jnp.min(jnp.where(is_max, col, ne), axis=1, keepdims=True)
        scores.append(m)
        indices.append(ind)
        vals = jnp.where(col == ind, neg_inf, vals)
    s = jnp.concatenate(scores, axis=1)
    idx = jnp.concatenate(indices, axis=1)
    e = jnp.exp(s - s[:, 0:1])
    sm_ref[...] = e / jnp.sum(e, axis=1, keepdims=True)
    idx_ref[...] = idx


@functools.partial(jax.jit, static_argnames=("bt",))
def _gate(x, wt, bt):
    t, d = x.shape
    ne = wt.shape[1]
    return pl.pallas_call(
        _gate_body,
        grid=(t // bt,),
        in_specs=[
            pl.BlockSpec((bt, d), lambda i: (i, 0)),
            pl.BlockSpec((d, ne), lambda i: (0, 0)),
        ],
        out_specs=[
            pl.BlockSpec((bt, _TOP_K), lambda i: (i, 0)),
            pl.BlockSpec((bt, _TOP_K), lambda i: (i, 0)),
        ],
        out_shape=[
            jax.ShapeDtypeStruct((t, _TOP_K), jnp.float32),
            jax.ShapeDtypeStruct((t, _TOP_K), jnp.int32),
        ],
    )(x, wt)


def kernel(x, W):
    sm, idx = _gate(x, W.T, bt=256)
    return (sm, idx)


# speedup vs baseline: 1.1789x; 1.1789x over previous
import functools

import jax
import jax.numpy as jnp
from jax.experimental import pallas as pl

_TOP_K = 8


def _gate_body(x_ref, w_ref, sm_ref, idx_ref):
    logits = jax.lax.dot_general(
        w_ref[...], x_ref[...],
        dimension_numbers=(((1,), (1,)), ((), ())),
        preferred_element_type=jnp.float32,
    )
    ne, bt = logits.shape
    row = jax.lax.broadcasted_iota(jnp.int32, (ne, bt), 0)
    bits = jax.lax.bitcast_convert_type(logits, jnp.int32)
    skey = bits ^ ((bits >> 31) & jnp.int32(0x7FFFFFFF))
    key = (skey & jnp.int32(~0x3F)) | (row ^ jnp.int32(0x3F))
    neg = jnp.int32(-(2**31))
    vals = key
    keys = []
    for _ in range(_TOP_K):
        m = jnp.max(vals, axis=0, keepdims=True)
        keys.append(m)
        vals = jnp.where(vals == m, neg, vals)
    k8 = jnp.concatenate(keys, axis=0).T
    idx = (k8 & jnp.int32(0x3F)) ^ jnp.int32(0x3F)
    st = k8 & jnp.int32(~0x3F)
    sbits = st ^ ((st >> 31) & jnp.int32(0x7FFFFFFF))
    s = jax.lax.bitcast_convert_type(sbits, jnp.float32)
    e = jnp.exp(s - s[:, 0:1])
    sm_ref[...] = e / jnp.sum(e, axis=1, keepdims=True)
    idx_ref[...] = idx


@functools.partial(jax.jit, static_argnames=("bt",))
def _gate(x, w, bt):
    t, d = x.shape
    ne = w.shape[0]
    return pl.pallas_call(
        _gate_body,
        grid=(t // bt,),
        in_specs=[
            pl.BlockSpec((bt, d), lambda i: (i, 0)),
            pl.BlockSpec((ne, d), lambda i: (0, 0)),
        ],
        out_specs=[
            pl.BlockSpec((bt, _TOP_K), lambda i: (i, 0)),
            pl.BlockSpec((bt, _TOP_K), lambda i: (i, 0)),
        ],
        out_shape=[
            jax.ShapeDtypeStruct((t, _TOP_K), jnp.float32),
            jax.ShapeDtypeStruct((t, _TOP_K), jnp.int32),
        ],
    )(x, w)


def kernel(x, W):
    sm, idx = _gate(x, W, bt=256)
    return (sm, idx)


# speedup vs baseline: 1.4431x; 1.2241x over previous
import functools

import jax
import jax.numpy as jnp
from jax.experimental import pallas as pl

_TOP_K = 8


def _gate_body(x_ref, w_ref, sm_ref, idx_ref):
    logits = jax.lax.dot_general(
        w_ref[...], x_ref[...],
        dimension_numbers=(((1,), (1,)), ((), ())),
        preferred_element_type=jnp.float32,
    )
    ne, bt = logits.shape
    row = jax.lax.broadcasted_iota(jnp.int32, (ne, bt), 0)
    bits = jax.lax.bitcast_convert_type(logits, jnp.int32)
    skey = bits ^ ((bits >> 31) & jnp.int32(0x7FFFFFFF))
    key = (skey & jnp.int32(~0x3F)) | (row ^ jnp.int32(0x3F))
    neg = jnp.int32(-(2**31))
    vals = key
    keys = []
    for _ in range(_TOP_K):
        m = jnp.max(vals, axis=0, keepdims=True)
        keys.append(m)
        vals = jnp.where(vals == m, neg, vals)
    k8 = jnp.concatenate(keys, axis=0).T
    idx = (k8 & jnp.int32(0x3F)) ^ jnp.int32(0x3F)
    st = k8 & jnp.int32(~0x3F)
    sbits = st ^ ((st >> 31) & jnp.int32(0x7FFFFFFF))
    s = jax.lax.bitcast_convert_type(sbits, jnp.float32)
    e = jnp.exp(s - s[:, 0:1])
    sm_ref[...] = e / jnp.sum(e, axis=1, keepdims=True)
    idx_ref[...] = idx


@functools.partial(jax.jit, static_argnames=("bt",))
def _gate(x, w, bt):
    t, d = x.shape
    ne = w.shape[0]
    return pl.pallas_call(
        _gate_body,
        grid=(t // bt,),
        in_specs=[
            pl.BlockSpec((bt, d), lambda i: (i, 0)),
            pl.BlockSpec((ne, d), lambda i: (0, 0)),
        ],
        out_specs=[
            pl.BlockSpec((bt, _TOP_K), lambda i: (i, 0)),
            pl.BlockSpec((bt, _TOP_K), lambda i: (i, 0)),
        ],
        out_shape=[
            jax.ShapeDtypeStruct((t, _TOP_K), jnp.float32),
            jax.ShapeDtypeStruct((t, _TOP_K), jnp.int32),
        ],
    )(x, w)


def kernel(x, W):
    sm, idx = _gate(x, W, bt=512)
    return (sm, idx)


# speedup vs baseline: 1.5483x; 1.0729x over previous
import functools

import jax
import jax.numpy as jnp
from jax.experimental import pallas as pl

_TOP_K = 8


def _gate_body(x_ref, w_ref, sm_ref, idx_ref):
    logits = jax.lax.dot_general(
        w_ref[...], x_ref[...],
        dimension_numbers=(((1,), (1,)), ((), ())),
        preferred_element_type=jnp.float32,
    )
    ne, bt = logits.shape
    row = jax.lax.broadcasted_iota(jnp.int32, (ne, bt), 0)
    bits = jax.lax.bitcast_convert_type(logits, jnp.int32)
    skey = bits ^ ((bits >> 31) & jnp.int32(0x7FFFFFFF))
    key = (skey & jnp.int32(~0x3F)) | (row ^ jnp.int32(0x3F))
    neg = jnp.int32(-(2**31))
    vals = key
    keys = []
    for _ in range(_TOP_K):
        m = jnp.max(vals, axis=0, keepdims=True)
        keys.append(m)
        vals = jnp.where(vals == m, neg, vals)
    k8 = jnp.concatenate(keys, axis=0).T
    idx = (k8 & jnp.int32(0x3F)) ^ jnp.int32(0x3F)
    st = k8 & jnp.int32(~0x3F)
    sbits = st ^ ((st >> 31) & jnp.int32(0x7FFFFFFF))
    s = jax.lax.bitcast_convert_type(sbits, jnp.float32)
    e = jnp.exp(s - s[:, 0:1])
    sm_ref[...] = e / jnp.sum(e, axis=1, keepdims=True)
    idx_ref[...] = idx


@functools.partial(jax.jit, static_argnames=("bt",))
def _gate(x, w, bt):
    t, d = x.shape
    ne = w.shape[0]
    return pl.pallas_call(
        _gate_body,
        grid=(t // bt,),
        in_specs=[
            pl.BlockSpec((bt, d), lambda i: (i, 0)),
            pl.BlockSpec((ne, d), lambda i: (0, 0)),
        ],
        out_specs=[
            pl.BlockSpec((bt, _TOP_K), lambda i: (i, 0)),
            pl.BlockSpec((bt, _TOP_K), lambda i: (i, 0)),
        ],
        out_shape=[
            jax.ShapeDtypeStruct((t, _TOP_K), jnp.float32),
            jax.ShapeDtypeStruct((t, _TOP_K), jnp.int32),
        ],
    )(x, w)


def kernel(x, W):
    sm, idx = _gate(x, W, bt=1024)
    return (sm, idx)
